# baseline (speedup 1.0000x reference)
import functools

import jax
import jax.numpy as jnp
from jax import lax
from jax.experimental import pallas as pl
from jax.experimental.pallas import tpu as pltpu
from jax.experimental.pallas import tpu_sc as plsc

_CHUNK = 128


def _build_embed(N, S, D):
    info = plsc.get_sparse_core_info()
    NW = info.num_cores * info.num_subcores
    n_per_w = N // NW
    n_chunks = n_per_w // _CHUNK
    rows_per_w = n_per_w // _CHUNK
    num_cores = info.num_cores

    mesh = plsc.VectorSubcoreMesh(core_axis_name="c", subcore_axis_name="s")

    @functools.partial(
        pl.kernel,
        mesh=mesh,
        out_type=jax.ShapeDtypeStruct((N, D), jnp.float32),
        scratch_types=[
            pltpu.VMEM((n_chunks, _CHUNK), jnp.int32),
            pltpu.VMEM((n_per_w, D), jnp.float32),
            pltpu.SemaphoreType.DMA,
        ],
        compiler_params=pltpu.CompilerParams(use_tc_tiling_on_sc=False),
    )
    def emb(ids_hbm, tok_hbm, pos_hbm, out_hbm, idx_v, buf_v, sem):
        wid = lax.axis_index("s") * num_cores + lax.axis_index("c")
        base = wid * n_per_w
        p0 = base % S
        pltpu.sync_copy(ids_hbm.at[pl.ds(wid * rows_per_w, rows_per_w)], idx_v)
        pltpu.sync_copy(pos_hbm.at[pl.ds(p0, n_per_w)], buf_v)
        copies = []
        for j in range(n_chunks):
            copies.append(
                pltpu.async_copy(
                    tok_hbm.at[idx_v.at[j]],
                    buf_v.at[pl.ds(j * _CHUNK, _CHUNK)],
                    sem,
                    add=True,
                )
            )
        for c in copies:
            c.wait()
        pltpu.sync_copy(buf_v, out_hbm.at[pl.ds(base, n_per_w)])

    return emb


def kernel(token_ids, token_table, pos_table):
    B, S = token_ids.shape
    V, D = token_table.shape
    N = B * S
    ids_2d = token_ids.reshape(N // _CHUNK, _CHUNK).astype(jnp.int32)
    emb = _build_embed(N, S, D)
    out = emb(ids_2d, token_table, pos_table)
    return out.reshape(B, S, D)
